# Initial kernel scaffold; baseline (speedup 1.0000x reference)
import jax
import jax.numpy as jnp
from jax.experimental import pallas as pl


def kernel(vectors, embed_table, W_rbf, W_msg, W_skip, W_prod, W_readout, senders, receivers, node_species):
    raise NotImplementedError("write your pallas kernel here")


# speedup vs baseline: 2.3932x; 2.3932x over previous
import functools

import jax
import jax.numpy as jnp
from jax import lax
from jax.experimental import pallas as pl
from jax.experimental.pallas import tpu as pltpu
from jax.experimental.pallas import tpu_sc as plsc

N_RADIAL = 8
ENVELOPE_P = 6
CUTOFF = 1.0
AVG_NUM_NEIGHBORS = 32.0
EPS = 1.0 / (1.0 + AVG_NUM_NEIGHBORS) ** 0.5

NW = 32
NC = 2
NS = 16
K_EDGE = 80
LANES = 16
IDX_BITS = 14


def _edge_tc_body(vref, w0ref, w1ref, o0ref, o1ref):
    v = vref[...]
    r = jnp.sqrt(jnp.sum(v * v, axis=1, keepdims=True) + 1e-12)
    r_safe = jnp.clip(r, 1e-6, None)
    be = v.shape[0]
    n = (lax.broadcasted_iota(jnp.int32, (be, N_RADIAL), 1) + 1
         ).astype(jnp.float32)
    rb = (jnp.sqrt(2.0 / CUTOFF) * jnp.sin(n * jnp.pi * r_safe / CUTOFF)
          / r_safe)
    u = r / CUTOFF
    p = float(ENVELOPE_P)
    env = (1.0
           - (p + 1.0) * (p + 2.0) / 2.0 * u ** ENVELOPE_P
           + p * (p + 2.0) * u ** (ENVELOPE_P + 1)
           - p * (p + 1.0) / 2.0 * u ** (ENVELOPE_P + 2))
    env = jnp.where(u < 1.0, env, 0.0)
    rbf = rb * env
    o0ref[...] = jnp.dot(rbf, w0ref[...], preferred_element_type=jnp.float32)
    o1ref[...] = jnp.dot(rbf, w1ref[...], preferred_element_type=jnp.float32)


def _radial_tc(vectors, w0, w1, be=2000):
    e, d = vectors.shape[0], w0.shape[1]
    return pl.pallas_call(
        _edge_tc_body,
        grid=(e // be,),
        in_specs=[
            pl.BlockSpec((be, 3), lambda i: (i, 0)),
            pl.BlockSpec((N_RADIAL, d), lambda i: (0, 0)),
            pl.BlockSpec((N_RADIAL, d), lambda i: (0, 0)),
        ],
        out_specs=[pl.BlockSpec((be, d), lambda i: (i, 0))] * 2,
        out_shape=[jax.ShapeDtypeStruct((e, d), jnp.float32)] * 2,
    )(vectors, w0, w1)


def _embed_tc_body(spref, tabref, oref):
    sp = spref[...]
    s = tabref.shape[0]
    bn = sp.shape[0]
    oh = (sp == lax.broadcasted_iota(jnp.int32, (bn, s), 1)).astype(jnp.float32)
    oref[...] = jnp.dot(oh, tabref[...], preferred_element_type=jnp.float32)


def _embed_tc(sp2d, table, bn=1000):
    n = sp2d.shape[0]
    s, d = table.shape
    return pl.pallas_call(
        _embed_tc_body,
        grid=(n // bn,),
        in_specs=[
            pl.BlockSpec((bn, 1), lambda i: (i, 0)),
            pl.BlockSpec((s, d), lambda i: (0, 0)),
        ],
        out_specs=pl.BlockSpec((bn, d), lambda i: (i, 0)),
        out_shape=jax.ShapeDtypeStruct((n, d), jnp.float32),
    )(sp2d, table)


def _make_sc_edge(n, d, ch):
    mesh = plsc.VectorSubcoreMesh(core_axis_name="c", subcore_axis_name="s")
    rows_full = 640
    rows_last = n - rows_full * (NS - 1)

    @functools.partial(
        pl.kernel,
        out_type=jax.ShapeDtypeStruct((NC, n, d), jnp.float32),
        mesh=mesh,
        scratch_types=[
            pltpu.VMEM((ch, K_EDGE), jnp.int32),
            pltpu.VMEM((K_EDGE,), jnp.int32),
            pltpu.VMEM((K_EDGE,), jnp.int32),
            pltpu.VMEM((K_EDGE, d), jnp.float32),
            pltpu.VMEM((K_EDGE, d), jnp.float32),
            pltpu.VMEM_SHARED((n, d), jnp.float32),
            pltpu.SemaphoreType.DMA,
            pltpu.SemaphoreType.DMA,
        ],
    )
    def sc_edge(nf_hbm, rad_hbm, idx_hbm, out_hbm,
                cidx_v, sidx_v, ridx_v, rows_v, rad_v, agg_sh, sem_g, sem_r):
        cid = lax.axis_index("c")
        sid = lax.axis_index("s")
        wid = sid * NC + cid
        r0 = sid * rows_full
        base_e = wid * ch * K_EDGE
        zero = jnp.zeros((LANES,), jnp.float32)
        mask = jnp.full((LANES,), (1 << IDX_BITS) - 1, jnp.int32)
        shift = jnp.full((LANES,), IDX_BITS, jnp.int32)

        pltpu.sync_copy(idx_hbm.at[wid], cidx_v)

        def zbody(i, _):
            for cc in range(d // LANES):
                rows_v[i, pl.ds(cc * LANES, LANES)] = zero
            return 0

        lax.fori_loop(0, K_EDGE, zbody, 0)

        @pl.when(sid < NS - 1)
        def _():
            for t in range(rows_full // K_EDGE):
                pltpu.sync_copy(rows_v,
                                agg_sh.at[pl.ds(r0 + t * K_EDGE, K_EDGE)])

        @pl.when(sid == NS - 1)
        def _():
            for t in range(rows_last // K_EDGE):
                pltpu.sync_copy(rows_v,
                                agg_sh.at[pl.ds(r0 + t * K_EDGE, K_EDGE)])

        plsc.subcore_barrier()

        def chunk(j, _):
            for c in range(K_EDGE // LANES):
                s = pl.ds(c * LANES, LANES)
                packed = cidx_v[j, s]
                sidx_v[s] = packed & mask
                ridx_v[s] = lax.shift_right_logical(packed, shift)
            g = pltpu.async_copy(nf_hbm.at[sidx_v], rows_v, sem_g)
            rc = pltpu.async_copy(
                rad_hbm.at[pl.ds(base_e + j * K_EDGE, K_EDGE)], rad_v, sem_r)
            g.wait()
            rc.wait()

            def mbody(i, _):
                for cc in range(d // LANES):
                    s = pl.ds(cc * LANES, LANES)
                    rows_v[i, s] = rows_v[i, s] * rad_v[i, s]
                return 0

            lax.fori_loop(0, K_EDGE, mbody, 0)
            pltpu.sync_copy(rows_v, agg_sh.at[ridx_v], add=True)
            return 0

        lax.fori_loop(0, ch, chunk, 0)
        plsc.subcore_barrier()

        @pl.when(sid < NS - 1)
        def _():
            pltpu.sync_copy(agg_sh.at[pl.ds(r0, rows_full)],
                            out_hbm.at[cid, pl.ds(r0, rows_full)])

        @pl.when(sid == NS - 1)
        def _():
            pltpu.sync_copy(agg_sh.at[pl.ds(r0, rows_last)],
                            out_hbm.at[cid, pl.ds(r0, rows_last)])

    return sc_edge


def _silu(x):
    return x * (1.0 / (1.0 + jnp.exp(-x)))


def _msg_block(aref, wmref, wpref):
    a = aref[...]
    agg = (a[0] + a[1]) * EPS
    h = _silu(jnp.dot(agg, wmref[...], preferred_element_type=jnp.float32)) * EPS
    hh = h + h * h + h * h * h
    return jnp.dot(hh, wpref[...], preferred_element_type=jnp.float32)


def _mid_tc_body(aref, wmref, wpref, oref):
    oref[...] = _msg_block(aref, wmref, wpref)


def _mid_tc(aggp, wm, wp, n, bn=1000):
    d = wm.shape[0]
    return pl.pallas_call(
        _mid_tc_body,
        grid=(n // bn,),
        in_specs=[
            pl.BlockSpec((NC, bn, d), lambda i: (0, i, 0)),
            pl.BlockSpec((d, d), lambda i: (0, 0)),
            pl.BlockSpec((d, d), lambda i: (0, 0)),
        ],
        out_specs=pl.BlockSpec((bn, d), lambda i: (i, 0)),
        out_shape=jax.ShapeDtypeStruct((n, d), jnp.float32),
    )(aggp, wm, wp)


def _final_tc_body(aref, nfref, spref, wmref, wpref, wskref, wr0ref, wr1ref,
                   oref):
    hp = _msg_block(aref, wmref, wpref)
    nf = nfref[...]
    sp = spref[...]
    nspec = wskref.shape[0]
    sc = jnp.zeros_like(hp)
    for s in range(nspec):
        zs = jnp.dot(nf, wskref[s], preferred_element_type=jnp.float32)
        sc = sc + jnp.where(sp == s, zs, 0.0)
    nf2 = hp + sc
    oref[...] = (jnp.dot(nf, wr0ref[...], preferred_element_type=jnp.float32)
                 + jnp.dot(nf2, wr1ref[...], preferred_element_type=jnp.float32))


def _final_tc(aggp, nf1, sp2d, wm, wp, wsk, wr0, wr1, bn=1000):
    n, d = nf1.shape
    nspec = wsk.shape[0]
    return pl.pallas_call(
        _final_tc_body,
        grid=(n // bn,),
        in_specs=[
            pl.BlockSpec((NC, bn, d), lambda i: (0, i, 0)),
            pl.BlockSpec((bn, d), lambda i: (i, 0)),
            pl.BlockSpec((bn, 1), lambda i: (i, 0)),
            pl.BlockSpec((d, d), lambda i: (0, 0)),
            pl.BlockSpec((d, d), lambda i: (0, 0)),
            pl.BlockSpec((nspec, d, d), lambda i: (0, 0, 0)),
            pl.BlockSpec((d, 1), lambda i: (0, 0)),
            pl.BlockSpec((d, 1), lambda i: (0, 0)),
        ],
        out_specs=pl.BlockSpec((bn, 1), lambda i: (i, 0)),
        out_shape=jax.ShapeDtypeStruct((n, 1), jnp.float32),
    )(aggp, nf1, sp2d, wm, wp, wsk, wr0, wr1)


def kernel(vectors, embed_table, W_rbf, W_msg, W_skip, W_prod, W_readout,
           senders, receivers, node_species):
    e = vectors.shape[0]
    n, d = node_species.shape[0], embed_table.shape[1]
    ew = e // NW
    ch = ew // K_EDGE

    snd = senders.astype(jnp.int32)
    rcv = receivers.astype(jnp.int32)
    packed = (snd | (rcv << IDX_BITS)).reshape(NW, ch, K_EDGE)
    sp2d = node_species.astype(jnp.int32).reshape(n, 1)

    radial0, radial1 = _radial_tc(vectors, W_rbf[0], W_rbf[1])
    nf0 = _embed_tc(sp2d, embed_table)

    sc_edge = _make_sc_edge(n, d, ch)
    aggp0 = sc_edge(nf0, radial0, packed)
    nf1 = _mid_tc(aggp0, W_msg[0], W_prod[0], n)
    aggp1 = sc_edge(nf1, radial1, packed)
    out = _final_tc(aggp1, nf1, sp2d, W_msg[1], W_prod[1], W_skip[1],
                    W_readout[0], W_readout[1])
    return out


# speedup vs baseline: 6.2318x; 2.6040x over previous
import functools

import jax
import jax.numpy as jnp
from jax import lax
from jax.experimental import pallas as pl
from jax.experimental.pallas import tpu as pltpu
from jax.experimental.pallas import tpu_sc as plsc

N_RADIAL = 8
ENVELOPE_P = 6
CUTOFF = 1.0
AVG_NUM_NEIGHBORS = 32.0
EPS = 1.0 / (1.0 + AVG_NUM_NEIGHBORS) ** 0.5

NW = 32
NC = 2
NS = 16
K_EDGE = 80
LANES = 16
IDX_BITS = 14


def _edge_tc_body(vref, w0ref, w1ref, o0ref, o1ref):
    v = vref[...]
    be = v.shape[1]
    r = jnp.sqrt(jnp.sum(v * v, axis=0, keepdims=True) + 1e-12)
    r_safe = jnp.clip(r, 1e-6, None)
    n = (lax.broadcasted_iota(jnp.int32, (N_RADIAL, be), 0) + 1
         ).astype(jnp.float32)
    sv = jnp.sin(n * (jnp.pi / CUTOFF) * r_safe)
    rb = jnp.sqrt(2.0 / CUTOFF) * sv / r_safe
    u = r / CUTOFF
    p = float(ENVELOPE_P)
    env = (1.0
           - (p + 1.0) * (p + 2.0) / 2.0 * u ** ENVELOPE_P
           + p * (p + 2.0) * u ** (ENVELOPE_P + 1)
           - p * (p + 1.0) / 2.0 * u ** (ENVELOPE_P + 2))
    env = jnp.where(u < 1.0, env, 0.0)
    rbf_t = rb * env
    dn = (((0,), (0,)), ((), ()))
    o0ref[...] = lax.dot_general(rbf_t, w0ref[...], dn,
                                 preferred_element_type=jnp.float32)
    o1ref[...] = lax.dot_general(rbf_t, w1ref[...], dn,
                                 preferred_element_type=jnp.float32)


def _radial_tc(vectors_t, w0, w1, be=3200):
    e, d = vectors_t.shape[1], w0.shape[1]
    return pl.pallas_call(
        _edge_tc_body,
        grid=(e // be,),
        in_specs=[
            pl.BlockSpec((3, be), lambda i: (0, i)),
            pl.BlockSpec((N_RADIAL, d), lambda i: (0, 0)),
            pl.BlockSpec((N_RADIAL, d), lambda i: (0, 0)),
        ],
        out_specs=[pl.BlockSpec((be, d), lambda i: (i, 0))] * 2,
        out_shape=[jax.ShapeDtypeStruct((e, d), jnp.float32)] * 2,
    )(vectors_t, w0, w1)


def _embed_tc_body(spref, tabref, oref):
    sp = spref[...]
    s = tabref.shape[0]
    bn = sp.shape[0]
    oh = (sp == lax.broadcasted_iota(jnp.int32, (bn, s), 1)).astype(jnp.float32)
    oref[...] = jnp.dot(oh, tabref[...], preferred_element_type=jnp.float32)


def _embed_tc(sp2d, table, bn=1000):
    n = sp2d.shape[0]
    s, d = table.shape
    return pl.pallas_call(
        _embed_tc_body,
        grid=(n // bn,),
        in_specs=[
            pl.BlockSpec((bn, 1), lambda i: (i, 0)),
            pl.BlockSpec((s, d), lambda i: (0, 0)),
        ],
        out_specs=pl.BlockSpec((bn, d), lambda i: (i, 0)),
        out_shape=jax.ShapeDtypeStruct((n, d), jnp.float32),
    )(sp2d, table)


def _make_sc_edge(n, d, ch):
    mesh = plsc.VectorSubcoreMesh(core_axis_name="c", subcore_axis_name="s")
    rows_full = 640
    rows_last = n - rows_full * (NS - 1)

    @functools.partial(
        pl.kernel,
        out_type=jax.ShapeDtypeStruct((NC, n, d), jnp.float32),
        mesh=mesh,
        scratch_types=[
            pltpu.VMEM((K_EDGE,), jnp.int32),
            pltpu.VMEM((K_EDGE,), jnp.int32),
            pltpu.VMEM((K_EDGE,), jnp.int32),
            pltpu.VMEM((K_EDGE,), jnp.int32),
            pltpu.VMEM((K_EDGE,), jnp.int32),
            pltpu.VMEM((K_EDGE,), jnp.int32),
            pltpu.VMEM((K_EDGE, d), jnp.float32),
            pltpu.VMEM((K_EDGE, d), jnp.float32),
            pltpu.VMEM((K_EDGE, d), jnp.float32),
            pltpu.VMEM((K_EDGE, d), jnp.float32),
            pltpu.VMEM_SHARED((n, d), jnp.float32),
            pltpu.SemaphoreType.DMA,
            pltpu.SemaphoreType.DMA,
            pltpu.SemaphoreType.DMA,
            pltpu.SemaphoreType.DMA,
        ],
    )
    def sc_edge(nf_hbm, rad_hbm, idx_hbm, out_hbm,
                cidx0, cidx1, sidx0, sidx1, ridx0, ridx1, rows0, rows1,
                rad0, rad1, agg_sh, semi0, semi1, semg0, semg1):
        cid = lax.axis_index("c")
        sid = lax.axis_index("s")
        wid = sid * NC + cid
        r0 = sid * rows_full
        base_e = wid * ch * K_EDGE
        zero = jnp.zeros((LANES,), jnp.float32)
        mask = jnp.full((LANES,), (1 << IDX_BITS) - 1, jnp.int32)
        shift = jnp.full((LANES,), IDX_BITS, jnp.int32)
        bufs = ((cidx0, sidx0, ridx0, rows0, rad0, semi0, semg0),
                (cidx1, sidx1, ridx1, rows1, rad1, semi1, semg1))

        def zbody(i, _):
            for cc in range(d // LANES):
                rows0[i, pl.ds(cc * LANES, LANES)] = zero
            return 0

        lax.fori_loop(0, K_EDGE, zbody, 0)

        @pl.when(sid < NS - 1)
        def _():
            for t in range(rows_full // K_EDGE):
                pltpu.sync_copy(rows0,
                                agg_sh.at[pl.ds(r0 + t * K_EDGE, K_EDGE)])

        @pl.when(sid == NS - 1)
        def _():
            for t in range(rows_last // K_EDGE):
                pltpu.sync_copy(rows0,
                                agg_sh.at[pl.ds(r0 + t * K_EDGE, K_EDGE)])

        plsc.subcore_barrier()

        def fire_idx(j, b):
            cidx, semi = bufs[b][0], bufs[b][5]
            pltpu.make_async_copy(
                idx_hbm.at[pl.ds(base_e + j * K_EDGE, K_EDGE)], cidx,
                semi).start()

        def wait_unpack(b):
            cidx, sidx, ridx, semi = bufs[b][0], bufs[b][1], bufs[b][2], \
                bufs[b][5]
            pltpu.make_async_copy(idx_hbm.at[pl.ds(base_e, K_EDGE)], cidx,
                                  semi).wait()
            for c in range(K_EDGE // LANES):
                s = pl.ds(c * LANES, LANES)
                packed = cidx[s]
                sidx[s] = packed & mask
                ridx[s] = lax.shift_right_logical(packed, shift)

        def fire_gr(j, b):
            sidx, rows, rad, semg = bufs[b][1], bufs[b][3], bufs[b][4], \
                bufs[b][6]
            pltpu.make_async_copy(nf_hbm.at[sidx], rows, semg).start()
            pltpu.make_async_copy(
                rad_hbm.at[pl.ds(base_e + j * K_EDGE, K_EDGE)], rad,
                semg).start()

        def drain_gr(b):
            sidx, rows, rad, semg = bufs[b][1], bufs[b][3], bufs[b][4], \
                bufs[b][6]
            pltpu.make_async_copy(nf_hbm.at[sidx], rows, semg).wait()
            pltpu.make_async_copy(rad_hbm.at[pl.ds(base_e, K_EDGE)], rad,
                                  semg).wait()

        def mult_scatter(b):
            ridx, rows, rad = bufs[b][2], bufs[b][3], bufs[b][4]

            def mbody(i, _):
                for cc in range(d // LANES):
                    s = pl.ds(cc * LANES, LANES)
                    rows[i, s] = rows[i, s] * rad[i, s]
                return 0

            lax.fori_loop(0, K_EDGE, mbody, 0)
            pltpu.sync_copy(rows, agg_sh.at[ridx], add=True)

        fire_idx(0, 0)
        wait_unpack(0)
        fire_gr(0, 0)
        fire_idx(1, 1)

        def pair(jj, _):
            for b in range(2):
                j = jj * 2 + b
                nb = (b + 1) % 2

                @pl.when(j < ch)
                def _():
                    @pl.when(j + 2 < ch)
                    def _():
                        fire_idx(j + 2, b)

                    @pl.when(j + 1 < ch)
                    def _():
                        wait_unpack(nb)
                        fire_gr(j + 1, nb)

                    drain_gr(b)
                    mult_scatter(b)
            return 0

        lax.fori_loop(0, (ch + 1) // 2, pair, 0)
        plsc.subcore_barrier()

        @pl.when(sid < NS - 1)
        def _():
            pltpu.sync_copy(agg_sh.at[pl.ds(r0, rows_full)],
                            out_hbm.at[cid, pl.ds(r0, rows_full)])

        @pl.when(sid == NS - 1)
        def _():
            pltpu.sync_copy(agg_sh.at[pl.ds(r0, rows_last)],
                            out_hbm.at[cid, pl.ds(r0, rows_last)])

    return sc_edge


def _silu(x):
    return x * (1.0 / (1.0 + jnp.exp(-x)))


def _msg_block(aref, wmref, wpref):
    a = aref[...]
    agg = (a[0] + a[1]) * EPS
    h = _silu(jnp.dot(agg, wmref[...], preferred_element_type=jnp.float32)) * EPS
    hh = h + h * h + h * h * h
    return jnp.dot(hh, wpref[...], preferred_element_type=jnp.float32)


def _mid_tc_body(aref, wmref, wpref, oref):
    oref[...] = _msg_block(aref, wmref, wpref)


def _mid_tc(aggp, wm, wp, n, bn=1000):
    d = wm.shape[0]
    return pl.pallas_call(
        _mid_tc_body,
        grid=(n // bn,),
        in_specs=[
            pl.BlockSpec((NC, bn, d), lambda i: (0, i, 0)),
            pl.BlockSpec((d, d), lambda i: (0, 0)),
            pl.BlockSpec((d, d), lambda i: (0, 0)),
        ],
        out_specs=pl.BlockSpec((bn, d), lambda i: (i, 0)),
        out_shape=jax.ShapeDtypeStruct((n, d), jnp.float32),
    )(aggp, wm, wp)


def _final_tc_body(aref, nfref, spref, wmref, wpref, wskref, wr0ref, wr1ref,
                   oref):
    hp = _msg_block(aref, wmref, wpref)
    nf = nfref[...]
    sp = spref[...]
    nspec = wskref.shape[0]
    sc = jnp.zeros_like(hp)
    for s in range(nspec):
        zs = jnp.dot(nf, wskref[s], preferred_element_type=jnp.float32)
        sc = sc + jnp.where(sp == s, zs, 0.0)
    nf2 = hp + sc
    oref[...] = (jnp.dot(nf, wr0ref[...], preferred_element_type=jnp.float32)
                 + jnp.dot(nf2, wr1ref[...], preferred_element_type=jnp.float32))


def _final_tc(aggp, nf1, sp2d, wm, wp, wsk, wr0, wr1, bn=1000):
    n, d = nf1.shape
    nspec = wsk.shape[0]
    return pl.pallas_call(
        _final_tc_body,
        grid=(n // bn,),
        in_specs=[
            pl.BlockSpec((NC, bn, d), lambda i: (0, i, 0)),
            pl.BlockSpec((bn, d), lambda i: (i, 0)),
            pl.BlockSpec((bn, 1), lambda i: (i, 0)),
            pl.BlockSpec((d, d), lambda i: (0, 0)),
            pl.BlockSpec((d, d), lambda i: (0, 0)),
            pl.BlockSpec((nspec, d, d), lambda i: (0, 0, 0)),
            pl.BlockSpec((d, 1), lambda i: (0, 0)),
            pl.BlockSpec((d, 1), lambda i: (0, 0)),
        ],
        out_specs=pl.BlockSpec((bn, 1), lambda i: (i, 0)),
        out_shape=jax.ShapeDtypeStruct((n, 1), jnp.float32),
    )(aggp, nf1, sp2d, wm, wp, wsk, wr0, wr1)


def kernel(vectors, embed_table, W_rbf, W_msg, W_skip, W_prod, W_readout,
           senders, receivers, node_species):
    e = vectors.shape[0]
    n, d = node_species.shape[0], embed_table.shape[1]
    ew = e // NW
    ch = ew // K_EDGE

    snd = senders.astype(jnp.int32)
    rcv = receivers.astype(jnp.int32)
    packed = snd | (rcv << IDX_BITS)
    sp2d = node_species.astype(jnp.int32).reshape(n, 1)

    radial0, radial1 = _radial_tc(vectors.T, W_rbf[0], W_rbf[1])
    nf0 = _embed_tc(sp2d, embed_table)

    sc_edge = _make_sc_edge(n, d, ch)
    aggp0 = sc_edge(nf0, radial0, packed)
    nf1 = _mid_tc(aggp0, W_msg[0], W_prod[0], n)
    aggp1 = sc_edge(nf1, radial1, packed)
    out = _final_tc(aggp1, nf1, sp2d, W_msg[1], W_prod[1], W_skip[1],
                    W_readout[0], W_readout[1])
    return out


# speedup vs baseline: 6.3113x; 1.0128x over previous
import functools

import jax
import jax.numpy as jnp
from jax import lax
from jax.experimental import pallas as pl
from jax.experimental.pallas import tpu as pltpu
from jax.experimental.pallas import tpu_sc as plsc

N_RADIAL = 8
ENVELOPE_P = 6
CUTOFF = 1.0
AVG_NUM_NEIGHBORS = 32.0
EPS = 1.0 / (1.0 + AVG_NUM_NEIGHBORS) ** 0.5

NW = 32
NC = 2
NS = 16
K_EDGE = 80
LANES = 16
IDX_BITS = 14


def _edge_tc_body(vref, w0ref, o0ref):
    v = vref[...]
    be = v.shape[1]
    r = jnp.sqrt(jnp.sum(v * v, axis=0, keepdims=True) + 1e-12)
    r_safe = jnp.clip(r, 1e-6, None)
    n = (lax.broadcasted_iota(jnp.int32, (N_RADIAL, be), 0) + 1
         ).astype(jnp.float32)
    sv = jnp.sin(n * (jnp.pi / CUTOFF) * r_safe)
    rb = jnp.sqrt(2.0 / CUTOFF) * sv / r_safe
    u = r / CUTOFF
    p = float(ENVELOPE_P)
    env = (1.0
           - (p + 1.0) * (p + 2.0) / 2.0 * u ** ENVELOPE_P
           + p * (p + 2.0) * u ** (ENVELOPE_P + 1)
           - p * (p + 1.0) / 2.0 * u ** (ENVELOPE_P + 2))
    env = jnp.where(u < 1.0, env, 0.0)
    rbf_t = rb * env
    dn = (((0,), (0,)), ((), ()))
    o0ref[...] = lax.dot_general(rbf_t, w0ref[...], dn,
                                 preferred_element_type=jnp.float32)


def _radial_tc(vectors_t, w0, be=3200):
    e, d = vectors_t.shape[1], w0.shape[1]
    return pl.pallas_call(
        _edge_tc_body,
        grid=(e // be,),
        in_specs=[
            pl.BlockSpec((3, be), lambda i: (0, i)),
            pl.BlockSpec((N_RADIAL, d), lambda i: (0, 0)),
        ],
        out_specs=pl.BlockSpec((be, d), lambda i: (i, 0)),
        out_shape=jax.ShapeDtypeStruct((e, d), jnp.float32),
    )(vectors_t, w0)


def _embed_tc_body(spref, tabref, oref):
    sp = spref[...]
    s = tabref.shape[0]
    bn = sp.shape[0]
    oh = (sp == lax.broadcasted_iota(jnp.int32, (bn, s), 1)).astype(jnp.float32)
    oref[...] = jnp.dot(oh, tabref[...], preferred_element_type=jnp.float32)


def _embed_tc(sp2d, table, bn=1000):
    n = sp2d.shape[0]
    s, d = table.shape
    return pl.pallas_call(
        _embed_tc_body,
        grid=(n // bn,),
        in_specs=[
            pl.BlockSpec((bn, 1), lambda i: (i, 0)),
            pl.BlockSpec((s, d), lambda i: (0, 0)),
        ],
        out_specs=pl.BlockSpec((bn, d), lambda i: (i, 0)),
        out_shape=jax.ShapeDtypeStruct((n, d), jnp.float32),
    )(sp2d, table)


def _make_sc_edge(n, d, ch):
    mesh = plsc.VectorSubcoreMesh(core_axis_name="c", subcore_axis_name="s")
    rows_full = 640
    rows_last = n - rows_full * (NS - 1)

    @functools.partial(
        pl.kernel,
        out_type=jax.ShapeDtypeStruct((NC, n, d), jnp.float32),
        mesh=mesh,
        scratch_types=[
            pltpu.VMEM((K_EDGE,), jnp.int32),
            pltpu.VMEM((K_EDGE,), jnp.int32),
            pltpu.VMEM((K_EDGE,), jnp.int32),
            pltpu.VMEM((K_EDGE,), jnp.int32),
            pltpu.VMEM((K_EDGE,), jnp.int32),
            pltpu.VMEM((K_EDGE,), jnp.int32),
            pltpu.VMEM((K_EDGE, d), jnp.float32),
            pltpu.VMEM((K_EDGE, d), jnp.float32),
            pltpu.VMEM((K_EDGE, d), jnp.float32),
            pltpu.VMEM((K_EDGE, d), jnp.float32),
            pltpu.VMEM_SHARED((n, d), jnp.float32),
            pltpu.SemaphoreType.DMA,
            pltpu.SemaphoreType.DMA,
            pltpu.SemaphoreType.DMA,
            pltpu.SemaphoreType.DMA,
        ],
    )
    def sc_edge(nf_hbm, rad_hbm, idx_hbm, out_hbm,
                cidx0, cidx1, sidx0, sidx1, ridx0, ridx1, rows0, rows1,
                rad0, rad1, agg_sh, semi0, semi1, semg0, semg1):
        cid = lax.axis_index("c")
        sid = lax.axis_index("s")
        wid = sid * NC + cid
        r0 = sid * rows_full
        base_e = wid * ch * K_EDGE
        zero = jnp.zeros((LANES,), jnp.float32)
        mask = jnp.full((LANES,), (1 << IDX_BITS) - 1, jnp.int32)
        shift = jnp.full((LANES,), IDX_BITS, jnp.int32)
        bufs = ((cidx0, sidx0, ridx0, rows0, rad0, semi0, semg0),
                (cidx1, sidx1, ridx1, rows1, rad1, semi1, semg1))

        def zbody(i, _):
            for cc in range(d // LANES):
                rows0[i, pl.ds(cc * LANES, LANES)] = zero
            return 0

        lax.fori_loop(0, K_EDGE, zbody, 0)

        @pl.when(sid < NS - 1)
        def _():
            for t in range(rows_full // K_EDGE):
                pltpu.sync_copy(rows0,
                                agg_sh.at[pl.ds(r0 + t * K_EDGE, K_EDGE)])

        @pl.when(sid == NS - 1)
        def _():
            for t in range(rows_last // K_EDGE):
                pltpu.sync_copy(rows0,
                                agg_sh.at[pl.ds(r0 + t * K_EDGE, K_EDGE)])

        plsc.subcore_barrier()

        def fire_idx(j, b):
            cidx, semi = bufs[b][0], bufs[b][5]
            pltpu.make_async_copy(
                idx_hbm.at[pl.ds(base_e + j * K_EDGE, K_EDGE)], cidx,
                semi).start()

        def wait_unpack(b):
            cidx, sidx, ridx, semi = bufs[b][0], bufs[b][1], bufs[b][2], \
                bufs[b][5]
            pltpu.make_async_copy(idx_hbm.at[pl.ds(base_e, K_EDGE)], cidx,
                                  semi).wait()
            for c in range(K_EDGE // LANES):
                s = pl.ds(c * LANES, LANES)
                packed = cidx[s]
                sidx[s] = packed & mask
                ridx[s] = lax.shift_right_logical(packed, shift)

        def fire_gr(j, b):
            sidx, rows, rad, semg = bufs[b][1], bufs[b][3], bufs[b][4], \
                bufs[b][6]
            pltpu.make_async_copy(nf_hbm.at[sidx], rows, semg).start()
            pltpu.make_async_copy(
                rad_hbm.at[pl.ds(base_e + j * K_EDGE, K_EDGE)], rad,
                semg).start()

        def drain_gr(b):
            sidx, rows, rad, semg = bufs[b][1], bufs[b][3], bufs[b][4], \
                bufs[b][6]
            pltpu.make_async_copy(nf_hbm.at[sidx], rows, semg).wait()
            pltpu.make_async_copy(rad_hbm.at[pl.ds(base_e, K_EDGE)], rad,
                                  semg).wait()

        def mult_scatter(b):
            ridx, rows, rad = bufs[b][2], bufs[b][3], bufs[b][4]

            def mbody(i, _):
                for cc in range(d // LANES):
                    s = pl.ds(cc * LANES, LANES)
                    rows[i, s] = rows[i, s] * rad[i, s]
                return 0

            lax.fori_loop(0, K_EDGE, mbody, 0)
            pltpu.sync_copy(rows, agg_sh.at[ridx], add=True)

        fire_idx(0, 0)
        wait_unpack(0)
        fire_gr(0, 0)
        fire_idx(1, 1)

        def pair(jj, _):
            for b in range(2):
                j = jj * 2 + b
                nb = (b + 1) % 2

                @pl.when(j < ch)
                def _():
                    @pl.when(j + 2 < ch)
                    def _():
                        fire_idx(j + 2, b)

                    @pl.when(j + 1 < ch)
                    def _():
                        wait_unpack(nb)
                        fire_gr(j + 1, nb)

                    drain_gr(b)
                    mult_scatter(b)
            return 0

        lax.fori_loop(0, (ch + 1) // 2, pair, 0)
        plsc.subcore_barrier()

        @pl.when(sid < NS - 1)
        def _():
            pltpu.sync_copy(agg_sh.at[pl.ds(r0, rows_full)],
                            out_hbm.at[cid, pl.ds(r0, rows_full)])

        @pl.when(sid == NS - 1)
        def _():
            pltpu.sync_copy(agg_sh.at[pl.ds(r0, rows_last)],
                            out_hbm.at[cid, pl.ds(r0, rows_last)])

    return sc_edge


def _silu(x):
    return x * (1.0 / (1.0 + jnp.exp(-x)))


def _msg_block(aref, wmref, wpref):
    a = aref[...]
    agg = (a[0] + a[1]) * EPS
    h = _silu(jnp.dot(agg, wmref[...], preferred_element_type=jnp.float32)) * EPS
    hh = h + h * h + h * h * h
    return jnp.dot(hh, wpref[...], preferred_element_type=jnp.float32)


def _mid_tc_body(aref, wmref, wpref, oref):
    oref[...] = _msg_block(aref, wmref, wpref)


def _mid_tc(aggp, wm, wp, n, bn=1000):
    d = wm.shape[0]
    return pl.pallas_call(
        _mid_tc_body,
        grid=(n // bn,),
        in_specs=[
            pl.BlockSpec((NC, bn, d), lambda i: (0, i, 0)),
            pl.BlockSpec((d, d), lambda i: (0, 0)),
            pl.BlockSpec((d, d), lambda i: (0, 0)),
        ],
        out_specs=pl.BlockSpec((bn, d), lambda i: (i, 0)),
        out_shape=jax.ShapeDtypeStruct((n, d), jnp.float32),
    )(aggp, wm, wp)


def _final_tc_body(aref, nfref, spref, wmref, wpref, wskref, wr0ref, wr1ref,
                   oref):
    hp = _msg_block(aref, wmref, wpref)
    nf = nfref[...]
    sp = spref[...]
    nspec = wskref.shape[0]
    sc = jnp.zeros_like(hp)
    for s in range(nspec):
        zs = jnp.dot(nf, wskref[s], preferred_element_type=jnp.float32)
        sc = sc + jnp.where(sp == s, zs, 0.0)
    nf2 = hp + sc
    oref[...] = (jnp.dot(nf, wr0ref[...], preferred_element_type=jnp.float32)
                 + jnp.dot(nf2, wr1ref[...], preferred_element_type=jnp.float32))


def _final_tc(aggp, nf1, sp2d, wm, wp, wsk, wr0, wr1, bn=1000):
    n, d = nf1.shape
    nspec = wsk.shape[0]
    return pl.pallas_call(
        _final_tc_body,
        grid=(n // bn,),
        in_specs=[
            pl.BlockSpec((NC, bn, d), lambda i: (0, i, 0)),
            pl.BlockSpec((bn, d), lambda i: (i, 0)),
            pl.BlockSpec((bn, 1), lambda i: (i, 0)),
            pl.BlockSpec((d, d), lambda i: (0, 0)),
            pl.BlockSpec((d, d), lambda i: (0, 0)),
            pl.BlockSpec((nspec, d, d), lambda i: (0, 0, 0)),
            pl.BlockSpec((d, 1), lambda i: (0, 0)),
            pl.BlockSpec((d, 1), lambda i: (0, 0)),
        ],
        out_specs=pl.BlockSpec((bn, 1), lambda i: (i, 0)),
        out_shape=jax.ShapeDtypeStruct((n, 1), jnp.float32),
    )(aggp, nf1, sp2d, wm, wp, wsk, wr0, wr1)


def kernel(vectors, embed_table, W_rbf, W_msg, W_skip, W_prod, W_readout,
           senders, receivers, node_species):
    e = vectors.shape[0]
    n, d = node_species.shape[0], embed_table.shape[1]
    ew = e // NW
    ch = ew // K_EDGE

    snd = senders.astype(jnp.int32)
    rcv = receivers.astype(jnp.int32)
    packed = snd | (rcv << IDX_BITS)
    sp2d = node_species.astype(jnp.int32).reshape(n, 1)

    vt = vectors.T
    radial0 = _radial_tc(vt, W_rbf[0])
    nf0 = _embed_tc(sp2d, embed_table)

    sc_edge = _make_sc_edge(n, d, ch)
    aggp0 = sc_edge(nf0, radial0, packed)
    radial1 = _radial_tc(vt, W_rbf[1])
    nf1 = _mid_tc(aggp0, W_msg[0], W_prod[0], n)
    aggp1 = sc_edge(nf1, radial1, packed)
    out = _final_tc(aggp1, nf1, sp2d, W_msg[1], W_prod[1], W_skip[1],
                    W_readout[0], W_readout[1])
    return out


# speedup vs baseline: 6.3851x; 1.0117x over previous
import functools

import jax
import jax.numpy as jnp
from jax import lax
from jax.experimental import pallas as pl
from jax.experimental.pallas import tpu as pltpu
from jax.experimental.pallas import tpu_sc as plsc

N_RADIAL = 8
ENVELOPE_P = 6
CUTOFF = 1.0
AVG_NUM_NEIGHBORS = 32.0
EPS = 1.0 / (1.0 + AVG_NUM_NEIGHBORS) ** 0.5

NW = 32

_LO_PERM = [32 * g + i for g in range(4) for i in range(16)]
_HI_PERM = [32 * g + 16 + i for g in range(4) for i in range(16)]
NC = 2
NS = 16
K_EDGE = 80
LANES = 16
IDX_BITS = 14


def _edge_tc_body(vref, wloref, whiref, o0ref):
    v = vref[...]
    be = v.shape[1]
    r = jnp.sqrt(jnp.sum(v * v, axis=0, keepdims=True) + 1e-12)
    r_safe = jnp.clip(r, 1e-6, None)
    n = (lax.broadcasted_iota(jnp.int32, (N_RADIAL, be), 0) + 1
         ).astype(jnp.float32)
    sv = jnp.sin(n * (jnp.pi / CUTOFF) * r_safe)
    rb = jnp.sqrt(2.0 / CUTOFF) * sv / r_safe
    u = r / CUTOFF
    p = float(ENVELOPE_P)
    env = (1.0
           - (p + 1.0) * (p + 2.0) / 2.0 * u ** ENVELOPE_P
           + p * (p + 2.0) * u ** (ENVELOPE_P + 1)
           - p * (p + 1.0) / 2.0 * u ** (ENVELOPE_P + 2))
    env = jnp.where(u < 1.0, env, 0.0)
    rbf_t = rb * env
    dn = (((0,), (0,)), ((), ()))
    alo = lax.dot_general(rbf_t, wloref[...], dn,
                          preferred_element_type=jnp.float32)
    ahi = lax.dot_general(rbf_t, whiref[...], dn,
                          preferred_element_type=jnp.float32)
    lo = lax.bitcast_convert_type(alo.astype(jnp.bfloat16),
                                  jnp.uint16).astype(jnp.int32)
    hi = lax.bitcast_convert_type(ahi.astype(jnp.bfloat16),
                                  jnp.uint16).astype(jnp.int32)
    o0ref[...] = lo | lax.shift_left(hi, 16)


def _radial_tc(vectors_t, wlo, whi, be=3200):
    e, dh = vectors_t.shape[1], wlo.shape[1]
    return pl.pallas_call(
        _edge_tc_body,
        grid=(e // be,),
        in_specs=[
            pl.BlockSpec((3, be), lambda i: (0, i)),
            pl.BlockSpec((N_RADIAL, dh), lambda i: (0, 0)),
            pl.BlockSpec((N_RADIAL, dh), lambda i: (0, 0)),
        ],
        out_specs=pl.BlockSpec((be, dh), lambda i: (i, 0)),
        out_shape=jax.ShapeDtypeStruct((e, dh), jnp.int32),
    )(vectors_t, wlo, whi)


def _embed_tc_body(spref, tabref, oref):
    sp = spref[...]
    s = tabref.shape[0]
    bn = sp.shape[0]
    oh = (sp == lax.broadcasted_iota(jnp.int32, (bn, s), 1)).astype(jnp.float32)
    oref[...] = jnp.dot(oh, tabref[...], preferred_element_type=jnp.float32)


def _embed_tc(sp2d, table, bn=1000):
    n = sp2d.shape[0]
    s, d = table.shape
    return pl.pallas_call(
        _embed_tc_body,
        grid=(n // bn,),
        in_specs=[
            pl.BlockSpec((bn, 1), lambda i: (i, 0)),
            pl.BlockSpec((s, d), lambda i: (0, 0)),
        ],
        out_specs=pl.BlockSpec((bn, d), lambda i: (i, 0)),
        out_shape=jax.ShapeDtypeStruct((n, d), jnp.float32),
    )(sp2d, table)


def _make_sc_edge(n, d, ch):
    mesh = plsc.VectorSubcoreMesh(core_axis_name="c", subcore_axis_name="s")
    rows_full = 640
    rows_last = n - rows_full * (NS - 1)

    @functools.partial(
        pl.kernel,
        out_type=jax.ShapeDtypeStruct((NC, n, d), jnp.float32),
        mesh=mesh,
        compiler_params=pltpu.CompilerParams(needs_layout_passes=False),
        scratch_types=[
            pltpu.VMEM((K_EDGE,), jnp.int32),
            pltpu.VMEM((K_EDGE,), jnp.int32),
            pltpu.VMEM((K_EDGE,), jnp.int32),
            pltpu.VMEM((K_EDGE,), jnp.int32),
            pltpu.VMEM((K_EDGE,), jnp.int32),
            pltpu.VMEM((K_EDGE,), jnp.int32),
            pltpu.VMEM((K_EDGE, d), jnp.float32),
            pltpu.VMEM((K_EDGE, d), jnp.float32),
            pltpu.VMEM((K_EDGE, d // 2), jnp.int32),
            pltpu.VMEM((K_EDGE, d // 2), jnp.int32),
            pltpu.VMEM_SHARED((n, d), jnp.float32),
            pltpu.SemaphoreType.DMA,
            pltpu.SemaphoreType.DMA,
            pltpu.SemaphoreType.DMA,
            pltpu.SemaphoreType.DMA,
        ],
    )
    def sc_edge(nf_hbm, rad_hbm, idx_hbm, out_hbm,
                cidx0, cidx1, sidx0, sidx1, ridx0, ridx1, rows0, rows1,
                rad0, rad1, agg_sh, semi0, semi1, semg0, semg1):
        cid = lax.axis_index("c")
        sid = lax.axis_index("s")
        wid = sid * NC + cid
        r0 = sid * rows_full
        base_e = wid * ch * K_EDGE
        zero = jnp.zeros((LANES,), jnp.float32)
        mask = jnp.full((LANES,), (1 << IDX_BITS) - 1, jnp.int32)
        shift = jnp.full((LANES,), IDX_BITS, jnp.int32)
        bufs = ((cidx0, sidx0, ridx0, rows0, rad0, semi0, semg0),
                (cidx1, sidx1, ridx1, rows1, rad1, semi1, semg1))

        def zbody(i, _):
            for cc in range(d // LANES):
                rows0[i, pl.ds(cc * LANES, LANES)] = zero
            return 0

        lax.fori_loop(0, K_EDGE, zbody, 0)

        @pl.when(sid < NS - 1)
        def _():
            for t in range(rows_full // K_EDGE):
                pltpu.sync_copy(rows0,
                                agg_sh.at[pl.ds(r0 + t * K_EDGE, K_EDGE)])

        @pl.when(sid == NS - 1)
        def _():
            for t in range(rows_last // K_EDGE):
                pltpu.sync_copy(rows0,
                                agg_sh.at[pl.ds(r0 + t * K_EDGE, K_EDGE)])

        plsc.subcore_barrier()

        def fire_idx(j, b):
            cidx, semi = bufs[b][0], bufs[b][5]
            pltpu.make_async_copy(
                idx_hbm.at[pl.ds(base_e + j * K_EDGE, K_EDGE)], cidx,
                semi).start()

        def wait_unpack(b):
            cidx, sidx, ridx, semi = bufs[b][0], bufs[b][1], bufs[b][2], \
                bufs[b][5]
            pltpu.make_async_copy(idx_hbm.at[pl.ds(base_e, K_EDGE)], cidx,
                                  semi).wait()
            for c in range(K_EDGE // LANES):
                s = pl.ds(c * LANES, LANES)
                packed = cidx[s]
                sidx[s] = packed & mask
                ridx[s] = lax.shift_right_logical(packed, shift)

        def fire_gr(j, b):
            sidx, rows, rad, semg = bufs[b][1], bufs[b][3], bufs[b][4], \
                bufs[b][6]
            pltpu.make_async_copy(nf_hbm.at[sidx], rows, semg).start()
            pltpu.make_async_copy(
                rad_hbm.at[pl.ds(base_e + j * K_EDGE, K_EDGE)], rad,
                semg).start()

        def drain_gr(b):
            sidx, rows, rad, semg = bufs[b][1], bufs[b][3], bufs[b][4], \
                bufs[b][6]
            pltpu.make_async_copy(nf_hbm.at[sidx], rows, semg).wait()
            pltpu.make_async_copy(rad_hbm.at[pl.ds(base_e, K_EDGE)], rad,
                                  semg).wait()

        def mult_scatter(b):
            ridx, rows, rad = bufs[b][2], bufs[b][3], bufs[b][4]

            shl16 = jnp.full((LANES,), 16, jnp.int32)
            himask = jnp.full((LANES,), -65536, jnp.int32)

            def mbody(i, _):
                for cc in range(d // (2 * LANES)):
                    pv = rad[i, pl.ds(cc * LANES, LANES)]
                    a = plsc.bitcast(lax.shift_left(pv, shl16), jnp.float32)
                    bb = plsc.bitcast(pv & himask, jnp.float32)
                    s0 = pl.ds(cc * 2 * LANES, LANES)
                    s1 = pl.ds(cc * 2 * LANES + LANES, LANES)
                    rows[i, s0] = rows[i, s0] * a
                    rows[i, s1] = rows[i, s1] * bb
                return 0

            lax.fori_loop(0, K_EDGE, mbody, 0)
            pltpu.sync_copy(rows, agg_sh.at[ridx], add=True)

        fire_idx(0, 0)
        wait_unpack(0)
        fire_gr(0, 0)
        fire_idx(1, 1)

        def pair(jj, _):
            for b in range(2):
                j = jj * 2 + b
                nb = (b + 1) % 2

                @pl.when(j < ch)
                def _():
                    @pl.when(j + 2 < ch)
                    def _():
                        fire_idx(j + 2, b)

                    @pl.when(j + 1 < ch)
                    def _():
                        wait_unpack(nb)
                        fire_gr(j + 1, nb)

                    drain_gr(b)
                    mult_scatter(b)
            return 0

        lax.fori_loop(0, (ch + 1) // 2, pair, 0)
        plsc.subcore_barrier()

        @pl.when(sid < NS - 1)
        def _():
            pltpu.sync_copy(agg_sh.at[pl.ds(r0, rows_full)],
                            out_hbm.at[cid, pl.ds(r0, rows_full)])

        @pl.when(sid == NS - 1)
        def _():
            pltpu.sync_copy(agg_sh.at[pl.ds(r0, rows_last)],
                            out_hbm.at[cid, pl.ds(r0, rows_last)])

    return sc_edge


def _silu(x):
    return x * (1.0 / (1.0 + jnp.exp(-x)))


def _msg_block(aref, wmref, wpref):
    a = aref[...]
    agg = (a[0] + a[1]) * EPS
    h = _silu(jnp.dot(agg, wmref[...], preferred_element_type=jnp.float32)) * EPS
    hh = h + h * h + h * h * h
    return jnp.dot(hh, wpref[...], preferred_element_type=jnp.float32)


def _mid_tc_body(aref, wmref, wpref, oref):
    oref[...] = _msg_block(aref, wmref, wpref)


def _mid_tc(aggp, wm, wp, n, bn=1000):
    d = wm.shape[0]
    return pl.pallas_call(
        _mid_tc_body,
        grid=(n // bn,),
        in_specs=[
            pl.BlockSpec((NC, bn, d), lambda i: (0, i, 0)),
            pl.BlockSpec((d, d), lambda i: (0, 0)),
            pl.BlockSpec((d, d), lambda i: (0, 0)),
        ],
        out_specs=pl.BlockSpec((bn, d), lambda i: (i, 0)),
        out_shape=jax.ShapeDtypeStruct((n, d), jnp.float32),
    )(aggp, wm, wp)


def _final_tc_body(aref, nfref, spref, wmref, wpref, wskref, wr0ref, wr1ref,
                   oref):
    hp = _msg_block(aref, wmref, wpref)
    nf = nfref[...]
    sp = spref[...]
    nspec = wskref.shape[0]
    sc = jnp.zeros_like(hp)
    for s in range(nspec):
        zs = jnp.dot(nf, wskref[s], preferred_element_type=jnp.float32)
        sc = sc + jnp.where(sp == s, zs, 0.0)
    nf2 = hp + sc
    oref[...] = (jnp.dot(nf, wr0ref[...], preferred_element_type=jnp.float32)
                 + jnp.dot(nf2, wr1ref[...], preferred_element_type=jnp.float32))


def _final_tc(aggp, nf1, sp2d, wm, wp, wsk, wr0, wr1, bn=1000):
    n, d = nf1.shape
    nspec = wsk.shape[0]
    return pl.pallas_call(
        _final_tc_body,
        grid=(n // bn,),
        in_specs=[
            pl.BlockSpec((NC, bn, d), lambda i: (0, i, 0)),
            pl.BlockSpec((bn, d), lambda i: (i, 0)),
            pl.BlockSpec((bn, 1), lambda i: (i, 0)),
            pl.BlockSpec((d, d), lambda i: (0, 0)),
            pl.BlockSpec((d, d), lambda i: (0, 0)),
            pl.BlockSpec((nspec, d, d), lambda i: (0, 0, 0)),
            pl.BlockSpec((d, 1), lambda i: (0, 0)),
            pl.BlockSpec((d, 1), lambda i: (0, 0)),
        ],
        out_specs=pl.BlockSpec((bn, 1), lambda i: (i, 0)),
        out_shape=jax.ShapeDtypeStruct((n, 1), jnp.float32),
    )(aggp, nf1, sp2d, wm, wp, wsk, wr0, wr1)


def kernel(vectors, embed_table, W_rbf, W_msg, W_skip, W_prod, W_readout,
           senders, receivers, node_species):
    e = vectors.shape[0]
    n, d = node_species.shape[0], embed_table.shape[1]
    ew = e // NW
    ch = ew // K_EDGE

    snd = senders.astype(jnp.int32)
    rcv = receivers.astype(jnp.int32)
    packed = snd | (rcv << IDX_BITS)
    sp2d = node_species.astype(jnp.int32).reshape(n, 1)

    vt = vectors.T
    lo_p = jnp.asarray(_LO_PERM, jnp.int32)
    hi_p = jnp.asarray(_HI_PERM, jnp.int32)
    radial0 = _radial_tc(vt, W_rbf[0][:, lo_p], W_rbf[0][:, hi_p])
    nf0 = _embed_tc(sp2d, embed_table)

    sc_edge = _make_sc_edge(n, d, ch)
    aggp0 = sc_edge(nf0, radial0, packed)
    radial1 = _radial_tc(vt, W_rbf[1][:, lo_p],
                         W_rbf[1][:, hi_p])
    nf1 = _mid_tc(aggp0, W_msg[0], W_prod[0], n)
    aggp1 = sc_edge(nf1, radial1, packed)
    out = _final_tc(aggp1, nf1, sp2d, W_msg[1], W_prod[1], W_skip[1],
                    W_readout[0], W_readout[1])
    return out


# speedup vs baseline: 6.3959x; 1.0017x over previous
import functools

import jax
import jax.numpy as jnp
from jax import lax
from jax.experimental import pallas as pl
from jax.experimental.pallas import tpu as pltpu
from jax.experimental.pallas import tpu_sc as plsc

N_RADIAL = 8
ENVELOPE_P = 6
CUTOFF = 1.0
AVG_NUM_NEIGHBORS = 32.0
EPS = 1.0 / (1.0 + AVG_NUM_NEIGHBORS) ** 0.5

NW = 32

_LO_PERM = [32 * g + i for g in range(4) for i in range(16)]
_HI_PERM = [32 * g + 16 + i for g in range(4) for i in range(16)]
NC = 2
NS = 16
K_EDGE = 80
LANES = 16
IDX_BITS = 14


def _edge_tc_body(vref, wloref, whiref, o0ref):
    v = vref[...]
    be = v.shape[1]
    r = jnp.sqrt(jnp.sum(v * v, axis=0, keepdims=True) + 1e-12)
    r_safe = jnp.clip(r, 1e-6, None)
    n = (lax.broadcasted_iota(jnp.int32, (N_RADIAL, be), 0) + 1
         ).astype(jnp.float32)
    sv = jnp.sin(n * (jnp.pi / CUTOFF) * r_safe)
    rb = jnp.sqrt(2.0 / CUTOFF) * sv / r_safe
    u = r / CUTOFF
    p = float(ENVELOPE_P)
    env = (1.0
           - (p + 1.0) * (p + 2.0) / 2.0 * u ** ENVELOPE_P
           + p * (p + 2.0) * u ** (ENVELOPE_P + 1)
           - p * (p + 1.0) / 2.0 * u ** (ENVELOPE_P + 2))
    env = jnp.where(u < 1.0, env, 0.0)
    rbf_t = rb * env
    dn = (((0,), (0,)), ((), ()))
    alo = lax.dot_general(rbf_t, wloref[...], dn,
                          preferred_element_type=jnp.float32)
    ahi = lax.dot_general(rbf_t, whiref[...], dn,
                          preferred_element_type=jnp.float32)
    lo = lax.bitcast_convert_type(alo.astype(jnp.bfloat16),
                                  jnp.uint16).astype(jnp.int32)
    hi = lax.bitcast_convert_type(ahi.astype(jnp.bfloat16),
                                  jnp.uint16).astype(jnp.int32)
    o0ref[...] = lo | lax.shift_left(hi, 16)


def _radial_tc(vectors_t, wlo, whi, be=3200):
    e, dh = vectors_t.shape[1], wlo.shape[1]
    return pl.pallas_call(
        _edge_tc_body,
        grid=(e // be,),
        in_specs=[
            pl.BlockSpec((3, be), lambda i: (0, i)),
            pl.BlockSpec((N_RADIAL, dh), lambda i: (0, 0)),
            pl.BlockSpec((N_RADIAL, dh), lambda i: (0, 0)),
        ],
        out_specs=pl.BlockSpec((be, dh), lambda i: (i, 0)),
        out_shape=jax.ShapeDtypeStruct((e, dh), jnp.int32),
    )(vectors_t, wlo, whi)


def _embed_tc_body(spref, tabref, oref):
    sp = spref[...]
    s = tabref.shape[0]
    bn = sp.shape[0]
    oh = (sp == lax.broadcasted_iota(jnp.int32, (bn, s), 1)).astype(jnp.float32)
    oref[...] = jnp.dot(oh, tabref[...], preferred_element_type=jnp.float32)


def _embed_tc(sp2d, table, bn=1000):
    n = sp2d.shape[0]
    s, d = table.shape
    return pl.pallas_call(
        _embed_tc_body,
        grid=(n // bn,),
        in_specs=[
            pl.BlockSpec((bn, 1), lambda i: (i, 0)),
            pl.BlockSpec((s, d), lambda i: (0, 0)),
        ],
        out_specs=pl.BlockSpec((bn, d), lambda i: (i, 0)),
        out_shape=jax.ShapeDtypeStruct((n, d), jnp.float32),
    )(sp2d, table)


def _make_sc_edge(n, d, ch):
    mesh = plsc.VectorSubcoreMesh(core_axis_name="c", subcore_axis_name="s")
    rows_full = 640
    rows_last = n - rows_full * (NS - 1)

    @functools.partial(
        pl.kernel,
        out_type=jax.ShapeDtypeStruct((NC, n, d), jnp.float32),
        mesh=mesh,
        compiler_params=pltpu.CompilerParams(needs_layout_passes=False),
        scratch_types=[
            pltpu.VMEM((K_EDGE,), jnp.int32),
            pltpu.VMEM((K_EDGE,), jnp.int32),
            pltpu.VMEM((K_EDGE,), jnp.int32),
            pltpu.VMEM((K_EDGE,), jnp.int32),
            pltpu.VMEM((K_EDGE,), jnp.int32),
            pltpu.VMEM((K_EDGE,), jnp.int32),
            pltpu.VMEM((K_EDGE, d), jnp.float32),
            pltpu.VMEM((K_EDGE, d), jnp.float32),
            pltpu.VMEM((K_EDGE, d // 2), jnp.int32),
            pltpu.VMEM((K_EDGE, d // 2), jnp.int32),
            pltpu.VMEM_SHARED((n, d), jnp.float32),
            pltpu.SemaphoreType.DMA,
            pltpu.SemaphoreType.DMA,
            pltpu.SemaphoreType.DMA,
            pltpu.SemaphoreType.DMA,
            pltpu.SemaphoreType.DMA,
            pltpu.SemaphoreType.DMA,
        ],
    )
    def sc_edge(nf_hbm, rad_hbm, idx_hbm, out_hbm,
                cidx0, cidx1, sidx0, sidx1, ridx0, ridx1, rows0, rows1,
                rad0, rad1, agg_sh, semi0, semi1, semg0, semg1, sems0, sems1):
        cid = lax.axis_index("c")
        sid = lax.axis_index("s")
        wid = sid * NC + cid
        r0 = sid * rows_full
        base_e = wid * ch * K_EDGE
        zero = jnp.zeros((LANES,), jnp.float32)
        mask = jnp.full((LANES,), (1 << IDX_BITS) - 1, jnp.int32)
        shift = jnp.full((LANES,), IDX_BITS, jnp.int32)
        bufs = ((cidx0, sidx0, ridx0, rows0, rad0, semi0, semg0, sems0),
                (cidx1, sidx1, ridx1, rows1, rad1, semi1, semg1, sems1))

        def zbody(i, _):
            for cc in range(d // LANES):
                rows0[i, pl.ds(cc * LANES, LANES)] = zero
            return 0

        lax.fori_loop(0, K_EDGE, zbody, 0)

        @pl.when(sid < NS - 1)
        def _():
            for t in range(rows_full // K_EDGE):
                pltpu.sync_copy(rows0,
                                agg_sh.at[pl.ds(r0 + t * K_EDGE, K_EDGE)])

        @pl.when(sid == NS - 1)
        def _():
            for t in range(rows_last // K_EDGE):
                pltpu.sync_copy(rows0,
                                agg_sh.at[pl.ds(r0 + t * K_EDGE, K_EDGE)])

        plsc.subcore_barrier()

        def fire_idx(j, b):
            cidx, semi = bufs[b][0], bufs[b][5]
            pltpu.make_async_copy(
                idx_hbm.at[pl.ds(base_e + j * K_EDGE, K_EDGE)], cidx,
                semi).start()

        def wait_unpack(b):
            cidx, sidx, ridx, semi = bufs[b][0], bufs[b][1], bufs[b][2], \
                bufs[b][5]
            pltpu.make_async_copy(idx_hbm.at[pl.ds(base_e, K_EDGE)], cidx,
                                  semi).wait()
            for c in range(K_EDGE // LANES):
                s = pl.ds(c * LANES, LANES)
                packed = cidx[s]
                sidx[s] = packed & mask
                ridx[s] = lax.shift_right_logical(packed, shift)

        def fire_gr(j, b):
            sidx, rows, rad, semg = bufs[b][1], bufs[b][3], bufs[b][4], \
                bufs[b][6]
            pltpu.make_async_copy(nf_hbm.at[sidx], rows, semg).start()
            pltpu.make_async_copy(
                rad_hbm.at[pl.ds(base_e + j * K_EDGE, K_EDGE)], rad,
                semg).start()

        def drain_gr(b):
            sidx, rows, rad, semg = bufs[b][1], bufs[b][3], bufs[b][4], \
                bufs[b][6]
            pltpu.make_async_copy(nf_hbm.at[sidx], rows, semg).wait()
            pltpu.make_async_copy(rad_hbm.at[pl.ds(base_e, K_EDGE)], rad,
                                  semg).wait()

        shl16 = jnp.full((LANES,), 16, jnp.int32)
        himask = jnp.full((LANES,), -65536, jnp.int32)

        def mult(b):
            rows, rad = bufs[b][3], bufs[b][4]

            def mbody(ii, _):
                for q in range(4):
                    i = ii * 4 + q
                    for cc in range(d // (2 * LANES)):
                        pv = rad[i, pl.ds(cc * LANES, LANES)]
                        a = plsc.bitcast(lax.shift_left(pv, shl16),
                                         jnp.float32)
                        bb = plsc.bitcast(pv & himask, jnp.float32)
                        s0 = pl.ds(cc * 2 * LANES, LANES)
                        s1 = pl.ds(cc * 2 * LANES + LANES, LANES)
                        rows[i, s0] = rows[i, s0] * a
                        rows[i, s1] = rows[i, s1] * bb
                return 0

            lax.fori_loop(0, K_EDGE // 4, mbody, 0)

        def fire_scatter(b):
            ridx, rows, sems = bufs[b][2], bufs[b][3], bufs[b][7]
            pltpu.async_copy(rows, agg_sh.at[ridx], sems, add=True)

        def drain_scatter(b):
            ridx, rows, sems = bufs[b][2], bufs[b][3], bufs[b][7]
            pltpu.make_async_copy(rows, agg_sh.at[ridx], sems).wait()

        fire_idx(0, 0)
        wait_unpack(0)
        fire_gr(0, 0)
        fire_idx(1, 1)

        def pair(jj, _):
            for b in range(2):
                j = jj * 2 + b
                nb = (b + 1) % 2

                @pl.when(j < ch)
                def _():
                    @pl.when(j + 2 < ch)
                    def _():
                        fire_idx(j + 2, b)

                    @pl.when(j >= 1)
                    def _():
                        drain_scatter(nb)

                    @pl.when(j + 1 < ch)
                    def _():
                        wait_unpack(nb)
                        fire_gr(j + 1, nb)

                    drain_gr(b)
                    mult(b)
                    fire_scatter(b)
            return 0

        lax.fori_loop(0, (ch + 1) // 2, pair, 0)
        drain_scatter((ch - 1) % 2)
        plsc.subcore_barrier()

        @pl.when(sid < NS - 1)
        def _():
            pltpu.sync_copy(agg_sh.at[pl.ds(r0, rows_full)],
                            out_hbm.at[cid, pl.ds(r0, rows_full)])

        @pl.when(sid == NS - 1)
        def _():
            pltpu.sync_copy(agg_sh.at[pl.ds(r0, rows_last)],
                            out_hbm.at[cid, pl.ds(r0, rows_last)])

    return sc_edge


def _silu(x):
    return x * (1.0 / (1.0 + jnp.exp(-x)))


def _msg_block(aref, wmref, wpref):
    a = aref[...]
    agg = (a[0] + a[1]) * EPS
    h = _silu(jnp.dot(agg, wmref[...], preferred_element_type=jnp.float32)) * EPS
    hh = h + h * h + h * h * h
    return jnp.dot(hh, wpref[...], preferred_element_type=jnp.float32)


def _mid_tc_body(aref, wmref, wpref, oref):
    oref[...] = _msg_block(aref, wmref, wpref)


def _mid_tc(aggp, wm, wp, n, bn=1000):
    d = wm.shape[0]
    return pl.pallas_call(
        _mid_tc_body,
        grid=(n // bn,),
        in_specs=[
            pl.BlockSpec((NC, bn, d), lambda i: (0, i, 0)),
            pl.BlockSpec((d, d), lambda i: (0, 0)),
            pl.BlockSpec((d, d), lambda i: (0, 0)),
        ],
        out_specs=pl.BlockSpec((bn, d), lambda i: (i, 0)),
        out_shape=jax.ShapeDtypeStruct((n, d), jnp.float32),
    )(aggp, wm, wp)


def _final_tc_body(aref, nfref, spref, wmref, wpref, wskref, wr0ref, wr1ref,
                   oref):
    hp = _msg_block(aref, wmref, wpref)
    nf = nfref[...]
    sp = spref[...]
    nspec = wskref.shape[0]
    sc = jnp.zeros_like(hp)
    for s in range(nspec):
        zs = jnp.dot(nf, wskref[s], preferred_element_type=jnp.float32)
        sc = sc + jnp.where(sp == s, zs, 0.0)
    nf2 = hp + sc
    oref[...] = (jnp.dot(nf, wr0ref[...], preferred_element_type=jnp.float32)
                 + jnp.dot(nf2, wr1ref[...], preferred_element_type=jnp.float32))


def _final_tc(aggp, nf1, sp2d, wm, wp, wsk, wr0, wr1, bn=1000):
    n, d = nf1.shape
    nspec = wsk.shape[0]
    return pl.pallas_call(
        _final_tc_body,
        grid=(n // bn,),
        in_specs=[
            pl.BlockSpec((NC, bn, d), lambda i: (0, i, 0)),
            pl.BlockSpec((bn, d), lambda i: (i, 0)),
            pl.BlockSpec((bn, 1), lambda i: (i, 0)),
            pl.BlockSpec((d, d), lambda i: (0, 0)),
            pl.BlockSpec((d, d), lambda i: (0, 0)),
            pl.BlockSpec((nspec, d, d), lambda i: (0, 0, 0)),
            pl.BlockSpec((d, 1), lambda i: (0, 0)),
            pl.BlockSpec((d, 1), lambda i: (0, 0)),
        ],
        out_specs=pl.BlockSpec((bn, 1), lambda i: (i, 0)),
        out_shape=jax.ShapeDtypeStruct((n, 1), jnp.float32),
    )(aggp, nf1, sp2d, wm, wp, wsk, wr0, wr1)


def kernel(vectors, embed_table, W_rbf, W_msg, W_skip, W_prod, W_readout,
           senders, receivers, node_species):
    e = vectors.shape[0]
    n, d = node_species.shape[0], embed_table.shape[1]
    ew = e // NW
    ch = ew // K_EDGE

    snd = senders.astype(jnp.int32)
    rcv = receivers.astype(jnp.int32)
    packed = snd | (rcv << IDX_BITS)
    sp2d = node_species.astype(jnp.int32).reshape(n, 1)

    vt = vectors.T
    lo_p = jnp.asarray(_LO_PERM, jnp.int32)
    hi_p = jnp.asarray(_HI_PERM, jnp.int32)
    radial0 = _radial_tc(vt, W_rbf[0][:, lo_p], W_rbf[0][:, hi_p])
    nf0 = _embed_tc(sp2d, embed_table)

    sc_edge = _make_sc_edge(n, d, ch)
    aggp0 = sc_edge(nf0, radial0, packed)
    radial1 = _radial_tc(vt, W_rbf[1][:, lo_p],
                         W_rbf[1][:, hi_p])
    nf1 = _mid_tc(aggp0, W_msg[0], W_prod[0], n)
    aggp1 = sc_edge(nf1, radial1, packed)
    out = _final_tc(aggp1, nf1, sp2d, W_msg[1], W_prod[1], W_skip[1],
                    W_readout[0], W_readout[1])
    return out
